# baseline (speedup 1.0000x reference)
import functools

import jax
import jax.numpy as jnp
from jax import lax
from jax.experimental import pallas as pl
from jax.experimental.pallas import tpu as pltpu
from jax.experimental.pallas import tpu_sc as plsc

_N = 262144
_K = 64
_NC = 2
_NS = 16
_NW = _NC * _NS
_PER_W = _N // _NW
_NVEC = _PER_W // 16
_HALF = _NVEC // 2

_BSTEPS = 8
_BROWS = (_N // 128) // _BSTEPS


def _sc_colsums(flat_dist, assignments):
    mesh = plsc.VectorSubcoreMesh(core_axis_name="c", subcore_axis_name="s")

    @functools.partial(
        pl.kernel,
        mesh=mesh,
        out_type=jax.ShapeDtypeStruct((_NW * 128,), jnp.float32),
        scratch_types=[
            pltpu.VMEM((_PER_W,), jnp.int32),
            pltpu.VMEM((_PER_W,), jnp.int32),
            pltpu.VMEM((_PER_W,), jnp.float32),
            pltpu.VMEM((128,), jnp.float32),
            pltpu.SemaphoreType.DMA,
            pltpu.SemaphoreType.DMA,
        ],
    )
    def sc_kernel(dist_hbm, a_hbm, cs_out,
                  a_v, idx_v, val_v, acc_v, sem0, sem1):
        sid = lax.axis_index("s")
        wid = sid * _NC + lax.axis_index("c")
        base = wid * _PER_W

        pltpu.sync_copy(a_hbm.at[pl.ds(base, _PER_W)], a_v)

        lane = lax.iota(jnp.int32, 16)

        def build(g, carry):
            a16 = a_v[pl.ds(g * 16, 16)]
            s = base + g * 16
            ipart = (s >> 7) * 1024 + (s & 127)
            idx_v[pl.ds(g * 16, 16)] = (
                ((a16 >> 3) << 21) + ((a16 & 7) << 7) + (ipart + lane))
            return carry

        lax.fori_loop(0, _HALF, build, 0, unroll=4)
        cp0 = pltpu.async_copy(
            dist_hbm.at[idx_v.at[pl.ds(0, _PER_W // 2)]],
            val_v.at[pl.ds(0, _PER_W // 2)], sem0)
        lax.fori_loop(_HALF, _NVEC, build, 0, unroll=4)
        cp1 = pltpu.async_copy(
            dist_hbm.at[idx_v.at[pl.ds(_PER_W // 2, _PER_W // 2)]],
            val_v.at[pl.ds(_PER_W // 2, _PER_W // 2)], sem1)

        zf = jnp.zeros((16,), jnp.float32)

        def accum(g, carry):
            c0, c1, c2, c3 = carry
            b = g * 64
            c0 = c0 + val_v[pl.ds(b, 16)]
            c1 = c1 + val_v[pl.ds(b + 16, 16)]
            c2 = c2 + val_v[pl.ds(b + 32, 16)]
            c3 = c3 + val_v[pl.ds(b + 48, 16)]
            return (c0, c1, c2, c3)

        cp0.wait()
        acc = lax.fori_loop(0, _NVEC // 8, accum, (zf, zf, zf, zf),
                            unroll=2)
        cp1.wait()
        acc = lax.fori_loop(_NVEC // 8, _NVEC // 4, accum, acc, unroll=2)

        for c in range(4):
            acc_v[pl.ds(c * 16, 16)] = acc[c]
            acc_v[pl.ds(64 + c * 16, 16)] = zf
        pltpu.sync_copy(acc_v, cs_out.at[pl.ds(wid * 128, 128)])

    return sc_kernel(flat_dist, assignments)


def _bincount_body(a_ref, cnt_out, acc_ref):
    g = pl.program_id(0)

    @pl.when(g == 0)
    def _init():
        acc_ref[...] = jnp.zeros((_K, 128), jnp.int32)

    row_iota = lax.broadcasted_iota(jnp.int32, (_K, 128), 0)

    def body(r, acc):
        arow = a_ref[pl.ds(r, 1), :]
        return acc + (row_iota == arow).astype(jnp.int32)

    acc_ref[...] = lax.fori_loop(0, _BROWS, body, acc_ref[...], unroll=8)

    @pl.when(g == _BSTEPS - 1)
    def _fin():
        cnt_out[...] = acc_ref[...]


def _combine_body(cs_ref, cnt_ref, out_ref):
    cs = jnp.sum(cs_ref[...], axis=0, keepdims=True)[:, 0:_K]
    cnt = jnp.sum(cnt_ref[...], axis=1, keepdims=True)
    valid = cnt > 0
    cntf = jnp.maximum(cnt, 1).astype(jnp.float32)
    recip = jnp.where(valid, 1.0 / cntf, 0.0)
    total = jax.lax.dot_general(
        cs, recip, (((1,), (0,)), ((), ())),
        precision=jax.lax.Precision.HIGHEST,
        preferred_element_type=jnp.float32)
    n_valid = jnp.sum(valid.astype(jnp.float32))
    out_ref[...] = total / jnp.maximum(n_valid, 1.0)


def kernel(distances, assignments):
    flat = (distances.T.reshape(8, 8, 2048, 128)
            .transpose(0, 2, 1, 3).reshape(-1))
    a2 = assignments.reshape(_N // 128, 128)
    cs = _sc_colsums(flat, assignments)
    cnt = pl.pallas_call(
        _bincount_body,
        grid=(_BSTEPS,),
        in_specs=[pl.BlockSpec((_BROWS, 128), lambda g: (g, 0))],
        out_specs=pl.BlockSpec((_K, 128), lambda g: (0, 0)),
        out_shape=jax.ShapeDtypeStruct((_K, 128), jnp.int32),
        scratch_shapes=[pltpu.VMEM((_K, 128), jnp.int32)],
    )(a2)
    out = pl.pallas_call(
        _combine_body,
        out_shape=jax.ShapeDtypeStruct((1, 1), jnp.float32),
    )(cs.reshape(_NW, 128), cnt)
    return out[0, 0]


# speedup vs baseline: 1.0093x; 1.0093x over previous
import functools

import jax
import jax.numpy as jnp
from jax import lax
from jax.experimental import pallas as pl
from jax.experimental.pallas import tpu as pltpu
from jax.experimental.pallas import tpu_sc as plsc

_N = 262144
_K = 64
_NC = 2
_NS = 16
_NW = _NC * _NS
_PER_W = _N // _NW
_NVEC = _PER_W // 16
_HALF = _NVEC // 2

_BSTEPS = 8
_BROWS = (_N // 128) // _BSTEPS


def _sc_colsums(flat_dist, assignments):
    mesh = plsc.VectorSubcoreMesh(core_axis_name="c", subcore_axis_name="s")

    @functools.partial(
        pl.kernel,
        mesh=mesh,
        out_type=jax.ShapeDtypeStruct((_NW * 128,), jnp.float32),
        scratch_types=[
            pltpu.VMEM((_PER_W,), jnp.int32),
            pltpu.VMEM((_PER_W,), jnp.int32),
            pltpu.VMEM((_PER_W,), jnp.float32),
            pltpu.VMEM((128,), jnp.float32),
            pltpu.SemaphoreType.DMA,
            pltpu.SemaphoreType.DMA,
            pltpu.SemaphoreType.DMA,
            pltpu.SemaphoreType.DMA,
        ],
    )
    def sc_kernel(dist_hbm, a_hbm, cs_out,
                  a_v, idx_v, val_v, acc_v, sem0, sem1, sem2, sem3):
        sid = lax.axis_index("s")
        wid = sid * _NC + lax.axis_index("c")
        base = wid * _PER_W

        pltpu.sync_copy(a_hbm.at[pl.ds(base, _PER_W)], a_v)

        lane = lax.iota(jnp.int32, 16)

        def build(g, carry):
            a16 = a_v[pl.ds(g * 16, 16)]
            s = base + g * 16
            ipart = (s >> 7) * 1024 + (s & 127)
            idx_v[pl.ds(g * 16, 16)] = (
                ((a16 >> 3) << 21) + ((a16 & 7) << 7) + (ipart + lane))
            return carry

        sems = (sem0, sem1, sem2, sem3)
        quarter = _NVEC // 4
        qelems = _PER_W // 4
        cps = []
        for k in range(4):
            lax.fori_loop(k * quarter, (k + 1) * quarter, build, 0,
                          unroll=8)
            cps.append(pltpu.async_copy(
                dist_hbm.at[idx_v.at[pl.ds(k * qelems, qelems)]],
                val_v.at[pl.ds(k * qelems, qelems)], sems[k]))

        zf = jnp.zeros((16,), jnp.float32)

        def accum(g, carry):
            c0, c1, c2, c3 = carry
            b = g * 64
            c0 = c0 + val_v[pl.ds(b, 16)]
            c1 = c1 + val_v[pl.ds(b + 16, 16)]
            c2 = c2 + val_v[pl.ds(b + 32, 16)]
            c3 = c3 + val_v[pl.ds(b + 48, 16)]
            return (c0, c1, c2, c3)

        acc = (zf, zf, zf, zf)
        qgrp = _NVEC // 16
        for k in range(4):
            cps[k].wait()
            acc = lax.fori_loop(k * qgrp, (k + 1) * qgrp, accum, acc,
                                unroll=4)

        for c in range(4):
            acc_v[pl.ds(c * 16, 16)] = acc[c]
            acc_v[pl.ds(64 + c * 16, 16)] = zf
        pltpu.sync_copy(acc_v, cs_out.at[pl.ds(wid * 128, 128)])

    return sc_kernel(flat_dist, assignments)


def _bincount_body(a_ref, cnt_out, acc_ref):
    g = pl.program_id(0)

    @pl.when(g == 0)
    def _init():
        acc_ref[...] = jnp.zeros((_K, 128), jnp.int32)

    row_iota = lax.broadcasted_iota(jnp.int32, (_K, 128), 0)

    def body(r, acc):
        arow = a_ref[pl.ds(r, 1), :]
        return acc + (row_iota == arow).astype(jnp.int32)

    acc_ref[...] = lax.fori_loop(0, _BROWS, body, acc_ref[...], unroll=8)

    @pl.when(g == _BSTEPS - 1)
    def _fin():
        cnt_out[...] = acc_ref[...]


def _combine_body(cs_ref, cnt_ref, out_ref):
    cs = jnp.sum(cs_ref[...], axis=0, keepdims=True)[:, 0:_K]
    cnt = jnp.sum(cnt_ref[...], axis=1, keepdims=True)
    valid = cnt > 0
    cntf = jnp.maximum(cnt, 1).astype(jnp.float32)
    recip = jnp.where(valid, 1.0 / cntf, 0.0)
    total = jax.lax.dot_general(
        cs, recip, (((1,), (0,)), ((), ())),
        precision=jax.lax.Precision.HIGHEST,
        preferred_element_type=jnp.float32)
    n_valid = jnp.sum(valid.astype(jnp.float32))
    out_ref[...] = total / jnp.maximum(n_valid, 1.0)


def kernel(distances, assignments):
    flat = (distances.T.reshape(8, 8, 2048, 128)
            .transpose(0, 2, 1, 3).reshape(-1))
    a2 = assignments.reshape(_N // 128, 128)
    cs = _sc_colsums(flat, assignments)
    cnt = pl.pallas_call(
        _bincount_body,
        grid=(_BSTEPS,),
        in_specs=[pl.BlockSpec((_BROWS, 128), lambda g: (g, 0))],
        out_specs=pl.BlockSpec((_K, 128), lambda g: (0, 0)),
        out_shape=jax.ShapeDtypeStruct((_K, 128), jnp.int32),
        scratch_shapes=[pltpu.VMEM((_K, 128), jnp.int32)],
    )(a2)
    out = pl.pallas_call(
        _combine_body,
        out_shape=jax.ShapeDtypeStruct((1, 1), jnp.float32),
    )(cs.reshape(_NW, 128), cnt)
    return out[0, 0]
